# baseline (speedup 1.0000x reference)
import functools

import jax
import jax.numpy as jnp
from jax import lax
from jax.experimental import pallas as pl
from jax.experimental.pallas import tpu as pltpu
from jax.experimental.pallas import tpu_sc as plsc

NUM_FACTORS = 64
BATCH = 16384
NW = 32
BPW = BATCH // NW
CHUNK = 128
NCHUNK = BPW // CHUNK
LANES = 16


def _body(users_hbm, items_hbm, L_hbm, R_hbm, Lb_hbm, Rb_hbm, out_hbm,
          idx_u, idx_v, rows_u, rows_v, bias_u, bias_v, out_vmem, sem):
    cid = lax.axis_index("c")
    sid = lax.axis_index("s")
    wid = sid * 2 + cid

    pltpu.sync_copy(users_hbm.at[wid], idx_u)
    pltpu.sync_copy(items_hbm.at[wid], idx_v)

    copies = []
    for c in range(NCHUNK):
        lo = c * CHUNK
        copies.append(pltpu.async_copy(
            L_hbm.at[idx_u.at[c]], rows_u.at[pl.ds(lo, CHUNK), :], sem))
        copies.append(pltpu.async_copy(
            R_hbm.at[idx_v.at[c]], rows_v.at[pl.ds(lo, CHUNK), :], sem))
        copies.append(pltpu.async_copy(
            Lb_hbm.at[idx_u.at[c]], bias_u.at[pl.ds(lo, CHUNK)], sem))
        copies.append(pltpu.async_copy(
            Rb_hbm.at[idx_v.at[c]], bias_v.at[pl.ds(lo, CHUNK)], sem))
    for cp in copies:
        cp.wait()

    lane = lax.iota(jnp.int32, LANES)

    @pl.loop(0, BPW // LANES)
    def _group(g):
        base = g * LANES
        row_ids = base + lane
        acc = bias_u[pl.ds(base, LANES)] + bias_v[pl.ds(base, LANES)]
        for k in range(NUM_FACTORS):
            col = jnp.full((LANES,), k, dtype=jnp.int32)
            u = plsc.load_gather(rows_u, [row_ids, col])
            v = plsc.load_gather(rows_v, [row_ids, col])
            acc = acc + u * v
        out_vmem[pl.ds(base, LANES)] = acc

    pltpu.sync_copy(out_vmem, out_hbm.at[pl.ds(wid * BPW, BPW)])


@jax.jit
def _mf_score(users, items, L, R, Lb, Rb):
    mesh = plsc.VectorSubcoreMesh(
        core_axis_name="c", subcore_axis_name="s", num_cores=2, num_subcores=16)
    kern = pl.kernel(
        _body,
        out_type=jax.ShapeDtypeStruct((BATCH,), jnp.float32),
        mesh=mesh,
        scratch_types=[
            pltpu.VMEM((NCHUNK, CHUNK), jnp.int32),
            pltpu.VMEM((NCHUNK, CHUNK), jnp.int32),
            pltpu.VMEM((BPW, NUM_FACTORS), jnp.float32),
            pltpu.VMEM((BPW, NUM_FACTORS), jnp.float32),
            pltpu.VMEM((BPW,), jnp.float32),
            pltpu.VMEM((BPW,), jnp.float32),
            pltpu.VMEM((BPW,), jnp.float32),
            pltpu.SemaphoreType.DMA,
        ],
        compiler_params=pltpu.CompilerParams(
            needs_layout_passes=False, use_tc_tiling_on_sc=False),
    )
    return kern(users, items, L, R, Lb, Rb)


def kernel(minibatch, L, R, L_bias, R_bias):
    users = minibatch[:, 0].reshape(NW, NCHUNK, CHUNK)
    items = minibatch[:, 1].reshape(NW, NCHUNK, CHUNK)
    return _mf_score(users, items, L, R, L_bias[:, 0], R_bias[:, 0])


# speedup vs baseline: 1.2838x; 1.2838x over previous
import jax
import jax.numpy as jnp
from jax import lax
from jax.experimental import pallas as pl
from jax.experimental.pallas import tpu as pltpu
from jax.experimental.pallas import tpu_sc as plsc

NUM_ROWS = 1000000
NUM_FACTORS = 64
BATCH = 16384
NW = 32
BPW = BATCH // NW
LANES = 16
TILE = 8
GROUPS = BPW // LANES
STAGE = BPW // 128
CP = 32
NCH = BPW // CP
BCHUNK = 128
NBCHUNK = BPW // BCHUNK
BIAS_PAD = 1000064
BIAS_TILES = BIAS_PAD // 128


def _body(users_hbm, items_hbm, L_hbm, R_hbm, Lb_hbm, Rb_hbm, out_hbm,
          idx_u, idx_v, hi7_u, hi7_v, lo7_u, lo7_v,
          lo3_u, lo3_v, slab_u, slab_v, btile_u, btile_v, out_vmem,
          sem_r, sem_b):
    cid = lax.axis_index("c")
    sid = lax.axis_index("s")
    wid = sid * 2 + cid

    pltpu.sync_copy(users_hbm.at[wid], idx_u)
    pltpu.sync_copy(items_hbm.at[wid], idx_v)

    lane = lax.iota(jnp.int32, LANES)

    for j in range(GROUPS):
        r, o = divmod(j * LANES, 128)
        cr, co = divmod(j * LANES, BCHUNK)
        u = idx_u[r, pl.ds(o, LANES)]
        v = idx_v[r, pl.ds(o, LANES)]
        hi7_u[cr, pl.ds(co, LANES)] = lax.shift_right_logical(u, 7)
        hi7_v[cr, pl.ds(co, LANES)] = lax.shift_right_logical(v, 7)
        lo7_u[pl.ds(j * LANES, LANES)] = lax.bitwise_and(u, 127)
        lo7_v[pl.ds(j * LANES, LANES)] = lax.bitwise_and(v, 127)
        lo3_u[pl.ds(j * LANES, LANES)] = lax.bitwise_and(u, 7)
        lo3_v[pl.ds(j * LANES, LANES)] = lax.bitwise_and(v, 7)

    for cb in range(NBCHUNK):
        du = pltpu.async_copy(Lb_hbm.at[hi7_u.at[cb]], btile_u, sem_b)
        dv = pltpu.async_copy(Rb_hbm.at[hi7_v.at[cb]], btile_v, sem_b)
        du.wait()
        dv.wait()
        for h in range(BCHUNK // LANES):
            base = cb * BCHUNK + h * LANES
            pos16 = h * LANES + lane
            bl = (plsc.load_gather(btile_u, [pos16, lo7_u[pl.ds(base, LANES)]])
                  + plsc.load_gather(btile_v, [pos16, lo7_v[pl.ds(base, LANES)]]))
            out_vmem[pl.ds(base, LANES)] = bl

    for c in range(NCH):

        @pl.loop(0, CP // LANES)
        def _fire(g, c=c):
            base = c * CP + g * LANES
            r = lax.shift_right_logical(base, 7)
            o = lax.bitwise_and(base, 127)
            uvec = idx_u[r, pl.ds(o, LANES)]
            vvec = idx_v[r, pl.ds(o, LANES)]
            jbase = g * LANES * TILE
            for i in range(LANES):
                ub = pl.multiple_of(
                    lax.bitwise_and(uvec[i], jnp.int32(-TILE)), TILE)
                vb = pl.multiple_of(
                    lax.bitwise_and(vvec[i], jnp.int32(-TILE)), TILE)
                pltpu.async_copy(
                    L_hbm.at[pl.ds(ub, TILE), :],
                    slab_u.at[pl.ds(jbase + i * TILE, TILE), :], sem_r)
                pltpu.async_copy(
                    R_hbm.at[pl.ds(vb, TILE), :],
                    slab_v.at[pl.ds(jbase + i * TILE, TILE), :], sem_r)

        for slab in (slab_u, slab_v):
            pltpu.make_async_copy(
                L_hbm.at[pl.ds(0, CP * TILE), :], slab, sem_r).wait()

        @pl.loop(0, CP // LANES)
        def _dot(h, c=c):
            base = c * CP + h * LANES
            srow = (h * LANES + lane) * TILE
            ru = srow + lo3_u[pl.ds(base, LANES)]
            rv = srow + lo3_v[pl.ds(base, LANES)]
            acc = out_vmem[pl.ds(base, LANES)]
            for k in range(NUM_FACTORS):
                col = jnp.full((LANES,), k, dtype=jnp.int32)
                uu = plsc.load_gather(slab_u, [ru, col])
                vv = plsc.load_gather(slab_v, [rv, col])
                acc = acc + uu * vv
            out_vmem[pl.ds(base, LANES)] = acc

    pltpu.sync_copy(out_vmem, out_hbm.at[pl.ds(wid * BPW, BPW)])


@jax.jit
def _mf_score(users, items, L, R, Lb, Rb):
    mesh = plsc.VectorSubcoreMesh(
        core_axis_name="c", subcore_axis_name="s", num_cores=2, num_subcores=16)
    kern = pl.kernel(
        _body,
        out_type=jax.ShapeDtypeStruct((BATCH,), jnp.float32),
        mesh=mesh,
        scratch_types=[
            pltpu.VMEM((STAGE, 128), jnp.int32),
            pltpu.VMEM((STAGE, 128), jnp.int32),
            pltpu.VMEM((NBCHUNK, BCHUNK), jnp.int32),
            pltpu.VMEM((NBCHUNK, BCHUNK), jnp.int32),
            pltpu.VMEM((BPW,), jnp.int32),
            pltpu.VMEM((BPW,), jnp.int32),
            pltpu.VMEM((BPW,), jnp.int32),
            pltpu.VMEM((BPW,), jnp.int32),
            pltpu.VMEM((CP * TILE, NUM_FACTORS), jnp.float32),
            pltpu.VMEM((CP * TILE, NUM_FACTORS), jnp.float32),
            pltpu.VMEM((BCHUNK, 128), jnp.float32),
            pltpu.VMEM((BCHUNK, 128), jnp.float32),
            pltpu.VMEM((BPW,), jnp.float32),
            pltpu.SemaphoreType.DMA,
            pltpu.SemaphoreType.DMA,
        ],
        compiler_params=pltpu.CompilerParams(needs_layout_passes=False),
    )
    return kern(users, items, L, R, Lb, Rb)


def kernel(minibatch, L, R, L_bias, R_bias):
    users = minibatch[:, 0].reshape(NW, STAGE, 128)
    items = minibatch[:, 1].reshape(NW, STAGE, 128)
    Lb = jnp.pad(L_bias[:, 0], (0, BIAS_PAD - NUM_ROWS)).reshape(BIAS_TILES, 128)
    Rb = jnp.pad(R_bias[:, 0], (0, BIAS_PAD - NUM_ROWS)).reshape(BIAS_TILES, 128)
    return _mf_score(users, items, L, R, Lb, Rb)


# speedup vs baseline: 1.2921x; 1.0065x over previous
import jax
import jax.numpy as jnp
from jax import lax
from jax.experimental import pallas as pl
from jax.experimental.pallas import tpu as pltpu
from jax.experimental.pallas import tpu_sc as plsc

NUM_ROWS = 1000000
NUM_FACTORS = 64
BATCH = 16384
NW = 32
BPW = BATCH // NW
LANES = 16
TILE = 8
GROUPS = BPW // LANES
STAGE = BPW // 128
CP = 32
NCH = BPW // CP


def _body(users_hbm, items_hbm, L_hbm, R_hbm, bu_hbm, bv_hbm, out_hbm,
          idx_u, idx_v, bias_u, bias_v, lo3_u, lo3_v, slab_u, slab_v,
          out_vmem, sem_r):
    cid = lax.axis_index("c")
    sid = lax.axis_index("s")
    wid = sid * 2 + cid

    pltpu.sync_copy(users_hbm.at[wid], idx_u)
    pltpu.sync_copy(items_hbm.at[wid], idx_v)
    pltpu.sync_copy(bu_hbm.at[wid], bias_u)
    pltpu.sync_copy(bv_hbm.at[wid], bias_v)

    lane = lax.iota(jnp.int32, LANES)

    for j in range(GROUPS):
        r, o = divmod(j * LANES, 128)
        u = idx_u[r, pl.ds(o, LANES)]
        v = idx_v[r, pl.ds(o, LANES)]
        lo3_u[pl.ds(j * LANES, LANES)] = lax.bitwise_and(u, 7)
        lo3_v[pl.ds(j * LANES, LANES)] = lax.bitwise_and(v, 7)

    for c in range(NCH):

        @pl.loop(0, CP // LANES)
        def _fire(g, c=c):
            base = c * CP + g * LANES
            r = lax.shift_right_logical(base, 7)
            o = lax.bitwise_and(base, 127)
            uvec = idx_u[r, pl.ds(o, LANES)]
            vvec = idx_v[r, pl.ds(o, LANES)]
            jbase = g * LANES * TILE
            for i in range(LANES):
                ub = pl.multiple_of(
                    lax.bitwise_and(uvec[i], jnp.int32(-TILE)), TILE)
                vb = pl.multiple_of(
                    lax.bitwise_and(vvec[i], jnp.int32(-TILE)), TILE)
                pltpu.async_copy(
                    L_hbm.at[pl.ds(ub, TILE), :],
                    slab_u.at[pl.ds(jbase + i * TILE, TILE), :], sem_r)
                pltpu.async_copy(
                    R_hbm.at[pl.ds(vb, TILE), :],
                    slab_v.at[pl.ds(jbase + i * TILE, TILE), :], sem_r)

        for slab in (slab_u, slab_v):
            pltpu.make_async_copy(
                L_hbm.at[pl.ds(0, CP * TILE), :], slab, sem_r).wait()

        @pl.loop(0, CP // LANES)
        def _dot(h, c=c):
            base = c * CP + h * LANES
            r = lax.shift_right_logical(base, 7)
            o = lax.bitwise_and(base, 127)
            srow = (h * LANES + lane) * TILE
            ru = srow + lo3_u[pl.ds(base, LANES)]
            rv = srow + lo3_v[pl.ds(base, LANES)]
            acc = bias_u[r, pl.ds(o, LANES)] + bias_v[r, pl.ds(o, LANES)]
            for k in range(NUM_FACTORS):
                col = jnp.full((LANES,), k, dtype=jnp.int32)
                uu = plsc.load_gather(slab_u, [ru, col])
                vv = plsc.load_gather(slab_v, [rv, col])
                acc = acc + uu * vv
            out_vmem[pl.ds(base, LANES)] = acc

    pltpu.sync_copy(out_vmem, out_hbm.at[pl.ds(wid * BPW, BPW)])


@jax.jit
def _mf_score(users, items, L, R, bu, bv):
    mesh = plsc.VectorSubcoreMesh(
        core_axis_name="c", subcore_axis_name="s", num_cores=2, num_subcores=16)
    kern = pl.kernel(
        _body,
        out_type=jax.ShapeDtypeStruct((BATCH,), jnp.float32),
        mesh=mesh,
        scratch_types=[
            pltpu.VMEM((STAGE, 128), jnp.int32),
            pltpu.VMEM((STAGE, 128), jnp.int32),
            pltpu.VMEM((STAGE, 128), jnp.float32),
            pltpu.VMEM((STAGE, 128), jnp.float32),
            pltpu.VMEM((BPW,), jnp.int32),
            pltpu.VMEM((BPW,), jnp.int32),
            pltpu.VMEM((CP * TILE, NUM_FACTORS), jnp.float32),
            pltpu.VMEM((CP * TILE, NUM_FACTORS), jnp.float32),
            pltpu.VMEM((BPW,), jnp.float32),
            pltpu.SemaphoreType.DMA,
        ],
        compiler_params=pltpu.CompilerParams(needs_layout_passes=False),
    )
    return kern(users, items, L, R, bu, bv)


def kernel(minibatch, L, R, L_bias, R_bias):
    users = minibatch[:, 0]
    items = minibatch[:, 1]
    bu = jnp.take(L_bias, users, axis=0)[:, 0].reshape(NW, STAGE, 128)
    bv = jnp.take(R_bias, items, axis=0)[:, 0].reshape(NW, STAGE, 128)
    return _mf_score(users.reshape(NW, STAGE, 128),
                     items.reshape(NW, STAGE, 128), L, R, bu, bv)


# speedup vs baseline: 3.0365x; 2.3501x over previous
import jax
import jax.numpy as jnp
from jax import lax
from jax.experimental import pallas as pl
from jax.experimental.pallas import tpu as pltpu
from jax.experimental.pallas import tpu_sc as plsc

NUM_ROWS = 1000000
NUM_FACTORS = 64
BATCH = 16384
NW = 32
BPW = BATCH // NW
LANES = 16
GROUPS = BPW // LANES
STAGE = BPW // 128
CH = 16
NCH = BPW // CH
KT = NUM_FACTORS // 8
CHUNK_BYTES = 2 * CH * KT * 8 * 16 * 4


def _fire_chunk(LT_hbm, RT_hbm, idx_u, idx_v, tu3, tv3, sems, b, c):
    base = c * CH
    r = lax.shift_right_logical(base, 7)
    o = lax.bitwise_and(base, 127)
    uvec = idx_u[r, pl.ds(o, LANES)]
    vvec = idx_v[r, pl.ds(o, LANES)]
    for i in range(CH):
        u16 = pl.multiple_of(lax.bitwise_and(uvec[i], jnp.int32(-16)), 16)
        v16 = pl.multiple_of(lax.bitwise_and(vvec[i], jnp.int32(-16)), 16)
        for kt in range(KT):
            pltpu.async_copy(
                LT_hbm.at[pl.ds(kt * 8, 8), pl.ds(u16, 16)],
                tu3.at[b, pl.ds(i * 8, 8), pl.ds(kt * 16, 16)], sems.at[b])
            pltpu.async_copy(
                RT_hbm.at[pl.ds(kt * 8, 8), pl.ds(v16, 16)],
                tv3.at[b, pl.ds(i * 8, 8), pl.ds(kt * 16, 16)], sems.at[b])


def _compute_chunk(bias_u, bias_v, lo4_u, lo4_v, tu3, tv3, out_vmem,
                   lane, b, c):
    base = c * CH
    r = lax.shift_right_logical(base, 7)
    o = lax.bitwise_and(base, 127)
    bvec = jnp.full((LANES,), 0, jnp.int32) + b
    prow = lane * 8
    cu = lo4_u[pl.ds(base, LANES)]
    cv = lo4_v[pl.ds(base, LANES)]
    acc = bias_u[r, pl.ds(o, LANES)] + bias_v[r, pl.ds(o, LANES)]
    for kt in range(KT):
        cbu = cu + kt * 16
        cbv = cv + kt * 16
        for ks in range(8):
            uu = plsc.load_gather(tu3, [bvec, prow + ks, cbu])
            vv = plsc.load_gather(tv3, [bvec, prow + ks, cbv])
            acc = acc + uu * vv
    out_vmem[pl.ds(base, LANES)] = acc


def _body(users_hbm, items_hbm, LT_hbm, RT_hbm, bu_hbm, bv_hbm, dummy_hbm,
          out_hbm, idx_u, idx_v, bias_u, bias_v, lo4_u, lo4_v,
          tu3, tv3, out_vmem, sems):
    cid = lax.axis_index("c")
    sid = lax.axis_index("s")
    wid = sid * 2 + cid

    pltpu.sync_copy(users_hbm.at[wid], idx_u)
    pltpu.sync_copy(items_hbm.at[wid], idx_v)
    pltpu.sync_copy(bu_hbm.at[wid], bias_u)
    pltpu.sync_copy(bv_hbm.at[wid], bias_v)

    lane = lax.iota(jnp.int32, LANES)

    for j in range(GROUPS):
        r, o = divmod(j * LANES, 128)
        u = idx_u[r, pl.ds(o, LANES)]
        v = idx_v[r, pl.ds(o, LANES)]
        lo4_u[pl.ds(j * LANES, LANES)] = lax.bitwise_and(u, 15)
        lo4_v[pl.ds(j * LANES, LANES)] = lax.bitwise_and(v, 15)

    @pl.loop(0, NCH + 1)
    def _pipe(c):
        b = lax.bitwise_and(c, 1)

        @pl.when(c < NCH)
        def _fire_cur():
            _fire_chunk(LT_hbm, RT_hbm, idx_u, idx_v, tu3, tv3, sems, b, c)

        @pl.when(c > 0)
        def _compute_prev():
            pb = 1 - b
            pltpu.make_async_copy(dummy_hbm, tu3.at[pb], sems.at[pb]).wait()
            pltpu.make_async_copy(dummy_hbm, tv3.at[pb], sems.at[pb]).wait()
            _compute_chunk(bias_u, bias_v, lo4_u, lo4_v, tu3, tv3,
                           out_vmem, lane, pb, c - 1)

    pltpu.sync_copy(out_vmem, out_hbm.at[pl.ds(wid * BPW, BPW)])


@jax.jit
def _mf_score(users, items, LT, RT, bu, bv, dummy):
    mesh = plsc.VectorSubcoreMesh(
        core_axis_name="c", subcore_axis_name="s", num_cores=2, num_subcores=16)
    kern = pl.kernel(
        _body,
        out_type=jax.ShapeDtypeStruct((BATCH,), jnp.float32),
        mesh=mesh,
        scratch_types=[
            pltpu.VMEM((STAGE, 128), jnp.int32),
            pltpu.VMEM((STAGE, 128), jnp.int32),
            pltpu.VMEM((STAGE, 128), jnp.float32),
            pltpu.VMEM((STAGE, 128), jnp.float32),
            pltpu.VMEM((BPW,), jnp.int32),
            pltpu.VMEM((BPW,), jnp.int32),
            pltpu.VMEM((2, CH * 8, 128), jnp.float32),
            pltpu.VMEM((2, CH * 8, 128), jnp.float32),
            pltpu.VMEM((BPW,), jnp.float32),
            pltpu.SemaphoreType.DMA((2,)),
        ],
        compiler_params=pltpu.CompilerParams(
            needs_layout_passes=False, use_tc_tiling_on_sc=True),
    )
    return kern(users, items, LT, RT, bu, bv, dummy)


def kernel(minibatch, L, R, L_bias, R_bias):
    users = minibatch[:, 0]
    items = minibatch[:, 1]
    bu = jnp.take(L_bias, users, axis=0)[:, 0].reshape(NW, STAGE, 128)
    bv = jnp.take(R_bias, items, axis=0)[:, 0].reshape(NW, STAGE, 128)
    dummy = jnp.zeros((CH * 8, 128), jnp.float32)
    return _mf_score(users.reshape(NW, STAGE, 128),
                     items.reshape(NW, STAGE, 128),
                     L.T, R.T, bu, bv, dummy)
